# baseline (speedup 1.0000x reference)
import functools

import jax
import jax.numpy as jnp
import numpy as np
from jax import lax
from jax.experimental import pallas as pl
from jax.experimental.pallas import tpu as pltpu
from jax.experimental.pallas import tpu_sc as plsc

_BS = 4096
_NOBJ = 21
_P = 26
_E = 32
_ROWS = _BS * _NOBJ
_NIDX = _ROWS * _P
_NBAND = _ROWS // 8
_LT = 7
_SLOTS = _LT * 32

_NC = 2
_NS = 16
_NW = _NC * _NS
_BPW = _NBAND // _NW
_NB = 7
_NCHUNK = _BPW // _NB
_BIN = _P * 8
_CIN = _NB * _BIN
_COUT = _NB * _SLOTS

assert _BPW * _NW == _NBAND and _NCHUNK * _NB == _BPW
assert _CIN % 8 == 0 and _COUT % 8 == 0 and _NCHUNK % 2 == 0


def _dst_pattern():
    j = np.arange(_BIN)
    s, p = j // _P, j % _P
    return ((p // 4) * 32 + s * 4 + p % 4).astype(np.int32)


def _pad_pattern():
    i = np.arange(16)
    return (192 + (i // 2) * 4 + 2 + i % 2).astype(np.int32)


_DPAT_NP = np.concatenate([_dst_pattern(), _pad_pattern()])

_S = 3
_NBAND_S = _NBAND // _S
_BPW_S = _NBAND_S // _NW
_NCHUNK_S = _BPW_S // _NB

assert _NBAND_S * _S == _NBAND and _BPW_S * _NW == _NBAND_S
assert _NCHUNK_S * _NB == _BPW_S and _NCHUNK_S % 2 == 0


def _make_sc_gather(slice_i):
    @functools.partial(
        pl.kernel,
        out_type=jax.ShapeDtypeStruct((_NBAND_S * _SLOTS, _E), jnp.float32),
        mesh=plsc.VectorSubcoreMesh(core_axis_name="c", subcore_axis_name="s"),
        scratch_types=[
            pltpu.VMEM((2, _CIN), jnp.int32),
            pltpu.VMEM((2, _COUT), jnp.int32),
            pltpu.VMEM((2, _COUT, _E), jnp.float32),
            pltpu.VMEM((_SLOTS,), jnp.int32),
            pltpu.SemaphoreType.DMA,
            pltpu.SemaphoreType.DMA,
            pltpu.SemaphoreType.DMA,
            pltpu.SemaphoreType.DMA,
        ],
        compiler_params=pltpu.CompilerParams(
            use_tc_tiling_on_sc=False, needs_layout_passes=False
        ),
    )
    def sc_gather(idx_hbm, table_hbm, dpat_hbm, out_hbm,
                  idx_in, idx_out, rows_v, dpat_v, g0, g1, s0, s1):
        wid = lax.axis_index("s") * _NC + lax.axis_index("c")
        ibase = (slice_i * _NBAND_S + wid * _BPW_S) * _BIN
        base = wid * _BPW_S * _SLOTS
        gsem = (g0, g1)
        ssem = (s0, s1)

        pltpu.sync_copy(dpat_hbm, dpat_v)
        dvec = [dpat_v[pl.ds(g * 16, 16)] for g in range(_SLOTS // 16)]

        def gather_start(i, b):
            pltpu.sync_copy(idx_hbm.at[pl.ds(ibase + i * _CIN, _CIN)], idx_in.at[b])
            for k in range(_NB):
                tbase = k * _SLOTS
                first = None
                for g in range(_BIN // 16):
                    v = idx_in[b, pl.ds(k * _BIN + g * 16, 16)]
                    if first is None:
                        first = v
                    plsc.store_scatter(idx_out.at[b], [dvec[g] + tbase], v)
                plsc.store_scatter(idx_out.at[b], [dvec[13] + tbase], first)
            pltpu.async_copy(table_hbm.at[idx_out.at[b]], rows_v.at[b], gsem[b])

        def gather_wait(b):
            pltpu.make_async_copy(table_hbm.at[idx_out.at[b]], rows_v.at[b], gsem[b]).wait()

        def store_start(i, b):
            pltpu.async_copy(rows_v.at[b], out_hbm.at[pl.ds(base + i * _COUT, _COUT)], ssem[b])

        def store_wait(i, b):
            pltpu.make_async_copy(rows_v.at[b], out_hbm.at[pl.ds(base + i * _COUT, _COUT)], ssem[b]).wait()

        gather_start(0, 0)
        gather_start(1, 1)

        def pair(j, carry):
            for b in range(2):
                i = 2 * j + b
                gather_wait(b)
                store_start(i, b)
                store_wait(i, b)
                gather_start(i + 2, b)
            return carry

        lax.fori_loop(0, (_NCHUNK_S - 2) // 2, pair, 0)

        for b in range(2):
            gather_wait(b)
            store_start(_NCHUNK_S - 2 + b, b)
        for b in range(2):
            store_wait(_NCHUNK_S - 2 + b, b)

    return sc_gather


_SC_GATHERS = [_make_sc_gather(s) for s in range(_S)]


_BB = 128
_RB = _BB * 8


def _tc_proj(g_ref, xs_ref, w_ref, b_ref, ma_ref, y_ref, m_ref):
    y = jnp.dot(g_ref[:, 0].reshape(_RB, 128), w_ref[0],
                preferred_element_type=jnp.float32)
    for lt in range(1, _LT):
        y = y + jnp.dot(g_ref[:, lt].reshape(_RB, 128), w_ref[lt],
                        preferred_element_type=jnp.float32)
    y = y + b_ref[...]
    pad = jnp.sum(xs_ref[...], axis=1, keepdims=True) == 0
    y_ref[...] = jnp.where(pad, ma_ref[...], y)
    m_ref[...] = pad.astype(jnp.int32)


def kernel(x, table, W, b, mark_absent):
    idx_flat = x.reshape(_NIDX)
    xs = x.reshape(_ROWS, _P)
    w4 = jnp.pad(W.T, ((0, _LT * 128 - _P * _E), (0, 0))).reshape(_LT, 128, _E)
    dpat = jnp.asarray(_DPAT_NP)
    b2 = b.reshape(1, _E)
    ma2 = mark_absent.reshape(1, _E)

    blocks_s = _NBAND_S // _BB
    ys, ms = [], []
    for s in range(_S):
        gathered = _SC_GATHERS[s](idx_flat, table, dpat)
        g4 = gathered.reshape(_NBAND_S, _LT, 8, 128)
        y_s, m_s = pl.pallas_call(
            _tc_proj,
            grid=(blocks_s,),
            in_specs=[
                pl.BlockSpec((_BB, _LT, 8, 128), lambda i: (i, 0, 0, 0)),
                pl.BlockSpec((_RB, _P), lambda i, s=s: (s * blocks_s + i, 0)),
                pl.BlockSpec((_LT, 128, _E), lambda i: (0, 0, 0)),
                pl.BlockSpec((1, _E), lambda i: (0, 0)),
                pl.BlockSpec((1, _E), lambda i: (0, 0)),
            ],
            out_specs=[
                pl.BlockSpec((_RB, _E), lambda i: (i, 0)),
                pl.BlockSpec((_RB, 1), lambda i: (i, 0)),
            ],
            out_shape=[
                jax.ShapeDtypeStruct((_NBAND_S * 8, _E), jnp.float32),
                jax.ShapeDtypeStruct((_NBAND_S * 8, 1), jnp.int32),
            ],
        )(g4, xs, w4, b2, ma2)
        ys.append(y_s)
        ms.append(m_s)

    y = jnp.concatenate(ys, axis=0)
    m = jnp.concatenate(ms, axis=0)
    obj_emb = y.reshape(_BS, _NOBJ, _E)
    padding = m.reshape(_BS, _NOBJ) != 0
    return obj_emb, padding


# speedup vs baseline: 1.0848x; 1.0848x over previous
import functools

import jax
import jax.numpy as jnp
import numpy as np
from jax import lax
from jax.experimental import pallas as pl
from jax.experimental.pallas import tpu as pltpu
from jax.experimental.pallas import tpu_sc as plsc

_BS = 4096
_NOBJ = 21
_P = 26
_E = 32
_ROWS = _BS * _NOBJ
_NIDX = _ROWS * _P
_NBAND = _ROWS // 8
_LT = 7
_SLOTS = _LT * 32

_NC = 2
_NS = 16
_NW = _NC * _NS
_BPW = _NBAND // _NW
_NB = 7
_NCHUNK = _BPW // _NB
_BIN = _P * 8
_CIN = _NB * _BIN
_COUT = _NB * _SLOTS

assert _BPW * _NW == _NBAND and _NCHUNK * _NB == _BPW
assert _CIN % 8 == 0 and _COUT % 8 == 0 and _NCHUNK % 2 == 0


def _dst_pattern():
    j = np.arange(_BIN)
    s, p = j // _P, j % _P
    return ((p // 4) * 32 + s * 4 + p % 4).astype(np.int32)


def _pad_pattern():
    i = np.arange(16)
    return (192 + (i // 2) * 4 + 2 + i % 2).astype(np.int32)


_DPAT_NP = np.concatenate([_dst_pattern(), _pad_pattern()])


@functools.partial(
    pl.kernel,
    out_type=jax.ShapeDtypeStruct((_NBAND * _SLOTS, _E), jnp.float32),
    mesh=plsc.VectorSubcoreMesh(core_axis_name="c", subcore_axis_name="s"),
    scratch_types=[
        pltpu.VMEM((2, _CIN), jnp.int32),
        pltpu.VMEM((2, _COUT), jnp.int32),
        pltpu.VMEM((2, _COUT, _E), jnp.float32),
        pltpu.VMEM((_SLOTS,), jnp.int32),
        pltpu.SemaphoreType.DMA,
        pltpu.SemaphoreType.DMA,
        pltpu.SemaphoreType.DMA,
        pltpu.SemaphoreType.DMA,
    ],
    compiler_params=pltpu.CompilerParams(
        use_tc_tiling_on_sc=False, needs_layout_passes=False
    ),
)
def _sc_gather(idx_hbm, table_hbm, dpat_hbm, out_hbm,
               idx_in, idx_out, rows_v, dpat_v, g0, g1, s0, s1):
    wid = lax.axis_index("s") * _NC + lax.axis_index("c")
    ibase = wid * _BPW * _BIN
    base = wid * _BPW * _SLOTS
    gsem = (g0, g1)
    ssem = (s0, s1)

    pltpu.sync_copy(dpat_hbm, dpat_v)
    dvec = [dpat_v[pl.ds(g * 16, 16)] for g in range(_SLOTS // 16)]

    def gather_start(i, b):
        pltpu.sync_copy(idx_hbm.at[pl.ds(ibase + i * _CIN, _CIN)], idx_in.at[b])
        for k in range(_NB):
            tbase = k * _SLOTS
            first = None
            for g in range(_BIN // 16):
                v = idx_in[b, pl.ds(k * _BIN + g * 16, 16)]
                if first is None:
                    first = v
                plsc.store_scatter(idx_out.at[b], [dvec[g] + tbase], v)
            plsc.store_scatter(idx_out.at[b], [dvec[13] + tbase], first)
        pltpu.async_copy(table_hbm.at[idx_out.at[b]], rows_v.at[b], gsem[b])

    def gather_wait(b):
        pltpu.make_async_copy(table_hbm.at[idx_out.at[b]], rows_v.at[b], gsem[b]).wait()

    def store_start(i, b):
        pltpu.async_copy(rows_v.at[b], out_hbm.at[pl.ds(base + i * _COUT, _COUT)], ssem[b])

    def store_wait(i, b):
        pltpu.make_async_copy(rows_v.at[b], out_hbm.at[pl.ds(base + i * _COUT, _COUT)], ssem[b]).wait()

    gather_start(0, 0)
    gather_start(1, 1)

    def pair(j, carry):
        for b in range(2):
            i = 2 * j + b
            gather_wait(b)
            store_start(i, b)
            store_wait(i, b)
            gather_start(i + 2, b)
        return carry

    lax.fori_loop(0, (_NCHUNK - 2) // 2, pair, 0)

    for b in range(2):
        gather_wait(b)
        store_start(_NCHUNK - 2 + b, b)
    for b in range(2):
        store_wait(_NCHUNK - 2 + b, b)


_BB = 128
_RB = _BB * 8


def _tc_proj(g_ref, xs_ref, w_ref, b_ref, ma_ref, y_ref, m_ref):
    y = jnp.dot(g_ref[:, 0].reshape(_RB, 128), w_ref[0],
                preferred_element_type=jnp.float32)
    for lt in range(1, _LT):
        y = y + jnp.dot(g_ref[:, lt].reshape(_RB, 128), w_ref[lt],
                        preferred_element_type=jnp.float32)
    y = y + b_ref[...]
    pad = jnp.sum(xs_ref[...], axis=1, keepdims=True) == 0
    y_ref[...] = jnp.where(pad, ma_ref[...], y)
    m_ref[...] = pad.astype(jnp.int32)


def kernel(x, table, W, b, mark_absent):
    idx_flat = x.reshape(_NIDX)
    gathered = _sc_gather(idx_flat, table, jnp.asarray(_DPAT_NP))

    g4 = gathered.reshape(_NBAND, _LT, 8, 128)
    xs = x.reshape(_ROWS, _P)
    w4 = jnp.pad(W.T, ((0, _LT * 128 - _P * _E), (0, 0))).reshape(_LT, 128, _E)
    y, m = pl.pallas_call(
        _tc_proj,
        grid=(_NBAND // _BB,),
        in_specs=[
            pl.BlockSpec((_BB, _LT, 8, 128), lambda i: (i, 0, 0, 0)),
            pl.BlockSpec((_RB, _P), lambda i: (i, 0)),
            pl.BlockSpec((_LT, 128, _E), lambda i: (0, 0, 0)),
            pl.BlockSpec((1, _E), lambda i: (0, 0)),
            pl.BlockSpec((1, _E), lambda i: (0, 0)),
        ],
        out_specs=[
            pl.BlockSpec((_RB, _E), lambda i: (i, 0)),
            pl.BlockSpec((_RB, 1), lambda i: (i, 0)),
        ],
        out_shape=[
            jax.ShapeDtypeStruct((_ROWS, _E), jnp.float32),
            jax.ShapeDtypeStruct((_ROWS, 1), jnp.int32),
        ],
    )(g4, xs, w4, b.reshape(1, _E), mark_absent.reshape(1, _E))

    obj_emb = y.reshape(_BS, _NOBJ, _E)
    padding = m.reshape(_BS, _NOBJ) != 0
    return obj_emb, padding
